# baseline (speedup 1.0000x reference)
import functools

import jax
import jax.numpy as jnp
from jax import lax
from jax.experimental import pallas as pl
from jax.experimental.pallas import tpu as pltpu
from jax.experimental.pallas import tpu_sc as plsc

_NC = 2
_NS = 16
_NW = _NC * _NS
_LANES = 16
_CHUNK = 8192
_UNROLL = 8


def _vec_loop(n_vec, body_u):
    n_outer = n_vec // _UNROLL
    if n_outer > 0:
        def outer(i, carry):
            b = i * (_LANES * _UNROLL)
            for u in range(_UNROLL):
                body_u(b + u * _LANES)
            return carry
        lax.fori_loop(0, n_outer, outer, 0, unroll=False)
    for u in range(n_vec % _UNROLL):
        body_u(n_outer * _LANES * _UNROLL + u * _LANES)


@functools.lru_cache(maxsize=None)
def _build(E, K):
    C = _CHUNK
    nc_full = K // C
    rem = K - nc_full * C
    n_vec_c = C // _LANES
    n_vec_t = (rem + _LANES - 1) // _LANES
    mesh = plsc.VectorSubcoreMesh(core_axis_name="c", subcore_axis_name="s")

    @functools.partial(
        pl.kernel,
        out_type=[
            jax.ShapeDtypeStruct((K,), jnp.int32),
            jax.ShapeDtypeStruct((K,), jnp.int32),
            jax.ShapeDtypeStruct((K,), jnp.float32),
        ],
        mesh=mesh,
        scratch_types=[
            pltpu.VMEM((C,), jnp.int32),
            pltpu.VMEM((C,), jnp.float32),
            pltpu.VMEM((C,), jnp.int32),
            pltpu.VMEM((C,), jnp.int32),
            pltpu.VMEM((C,), jnp.int32),
            pltpu.VMEM((C,), jnp.float32),
            pltpu.VMEM((C,), jnp.int32),
            pltpu.VMEM((C,), jnp.int32),
            pltpu.VMEM((_LANES,), jnp.float32),
            pltpu.SemaphoreType.DMA,
            pltpu.SemaphoreType.DMA,
            pltpu.SemaphoreType.DMA,
            pltpu.SemaphoreType.DMA,
        ],
    )
    def sc_kernel(vals_hbm, row0_hbm, row1_hbm, kept_hbm, inv_hbm,
                  out_r0_hbm, out_r1_hbm, out_vals_hbm,
                  idx_a, val_a, r0_a, r1_a,
                  idx_b, val_b, r0_b, r1_b,
                  inv_v, sga, sgb, swa, swb):
        w = lax.axis_index("s") * _NC + lax.axis_index("c")
        pltpu.sync_copy(inv_hbm, inv_v)
        inv = inv_v[...]

        def load_and_gather(base, n, bufs, sem):
            idx_v, val_v, r0_v, r1_v = bufs
            pltpu.sync_copy(kept_hbm.at[pl.ds(base, n)], idx_v.at[pl.ds(0, n)])
            cps = (
                pltpu.async_copy(vals_hbm.at[idx_v.at[pl.ds(0, n)]],
                                 val_v.at[pl.ds(0, n)], sem),
                pltpu.async_copy(row0_hbm.at[idx_v.at[pl.ds(0, n)]],
                                 r0_v.at[pl.ds(0, n)], sem),
                pltpu.async_copy(row1_hbm.at[idx_v.at[pl.ds(0, n)]],
                                 r1_v.at[pl.ds(0, n)], sem),
            )
            return cps

        def scale_and_write(base, n, n_vec, bufs, cps, wsem):
            _, val_v, r0_v, r1_v = bufs
            for cp in cps:
                cp.wait()
            wps = [
                pltpu.async_copy(r0_v.at[pl.ds(0, n)],
                                 out_r0_hbm.at[pl.ds(base, n)], wsem),
                pltpu.async_copy(r1_v.at[pl.ds(0, n)],
                                 out_r1_hbm.at[pl.ds(base, n)], wsem),
            ]

            def scale(s):
                val_v[pl.ds(s, _LANES)] = val_v[pl.ds(s, _LANES)] * inv
            _vec_loop(n_vec, scale)

            wps.append(pltpu.async_copy(val_v.at[pl.ds(0, n)],
                                        out_vals_hbm.at[pl.ds(base, n)], wsem))
            return wps

        bufs_a = (idx_a, val_a, r0_a, r1_a)
        bufs_b = (idx_b, val_b, r0_b, r1_b)

        def do_single(base, n, n_vec):
            cps = load_and_gather(base, n, bufs_a, sga)
            wps = scale_and_write(base, n, n_vec, bufs_a, cps, swa)
            for wp in wps:
                wp.wait()

        n_base = nc_full // _NW
        n_extra = nc_full % _NW
        n_w = n_base + jnp.where(w < n_extra, 1, 0)
        n_pairs = n_w // 2

        def pair_body(i, carry):
            b0 = (w + (2 * i) * _NW) * C
            b1 = (w + (2 * i + 1) * _NW) * C
            cps_a = load_and_gather(b0, C, bufs_a, sga)
            cps_b = load_and_gather(b1, C, bufs_b, sgb)
            wps_a = scale_and_write(b0, C, n_vec_c, bufs_a, cps_a, swa)
            wps_b = scale_and_write(b1, C, n_vec_c, bufs_b, cps_b, swb)
            for wp in wps_a + wps_b:
                wp.wait()
            return carry
        lax.fori_loop(0, n_pairs, pair_body, 0, unroll=False)

        @pl.when(n_w % 2 == 1)
        def _odd():
            do_single((w + (n_w - 1) * _NW) * C, C, n_vec_c)

        if rem > 0:
            @pl.when(w == _NW - 1)
            def _tail():
                do_single(nc_full * C, rem, n_vec_t)

    return sc_kernel


def kernel(vals, idxs, kept_idx, keepRate):
    E = vals.shape[0]
    K = kept_idx.shape[0]
    inv = jnp.full((_LANES,), 1.0, dtype=jnp.float32) / jnp.asarray(
        keepRate, dtype=jnp.float32)
    out_r0, out_r1, out_vals = _build(E, K)(
        vals, idxs[0], idxs[1], kept_idx, inv)
    return (jnp.stack([out_r0, out_r1]), out_vals)


# speedup vs baseline: 2.4607x; 2.4607x over previous
import functools

import jax
import jax.numpy as jnp
from jax import lax
from jax.experimental import pallas as pl
from jax.experimental.pallas import tpu as pltpu
from jax.experimental.pallas import tpu_sc as plsc

_NC = 2
_NS = 16
_NW = _NC * _NS
_LANES = 16
_CHUNK = 8192
_STAGE = 12288
_UNROLL = 8


def _vec_loop(n_vec, body_u):
    n_outer = n_vec // _UNROLL
    if n_outer > 0:
        def outer(i, carry):
            b = i * (_LANES * _UNROLL)
            for u in range(_UNROLL):
                body_u(b + u * _LANES)
            return carry
        lax.fori_loop(0, n_outer, outer, 0, unroll=False)
    for u in range(n_vec % _UNROLL):
        body_u(n_outer * _LANES * _UNROLL + u * _LANES)


@functools.lru_cache(maxsize=None)
def _build(E, K):
    C = _CHUNK
    S = _STAGE
    assert S <= E and S % 8 == 0 and E % 8 == 0
    nc_full = K // C
    rem = K - nc_full * C
    n_vec_c = C // _LANES
    n_vec_t = (rem + _LANES - 1) // _LANES
    mesh = plsc.VectorSubcoreMesh(core_axis_name="c", subcore_axis_name="s")

    @functools.partial(
        pl.kernel,
        out_type=[
            jax.ShapeDtypeStruct((K,), jnp.int32),
            jax.ShapeDtypeStruct((K,), jnp.int32),
            jax.ShapeDtypeStruct((K,), jnp.float32),
        ],
        mesh=mesh,
        compiler_params=pltpu.CompilerParams(needs_layout_passes=False),
        scratch_types=[
            pltpu.VMEM((C,), jnp.int32),
            pltpu.VMEM((S,), jnp.float32),
            pltpu.VMEM((S,), jnp.int32),
            pltpu.VMEM((S,), jnp.int32),
            pltpu.VMEM((C,), jnp.float32),
            pltpu.VMEM((C,), jnp.int32),
            pltpu.VMEM((C,), jnp.int32),
            pltpu.VMEM((_LANES,), jnp.float32),
            pltpu.SemaphoreType.DMA,
            pltpu.SemaphoreType.DMA,
        ],
    )
    def sc_kernel(vals_hbm, row0_hbm, row1_hbm, kept_hbm, inv_hbm,
                  out_r0_hbm, out_r1_hbm, out_vals_hbm,
                  idx_v, st_v, st_r0, st_r1,
                  ov_v, ov_r0, ov_r1, inv_v, sg, sw):
        w = lax.axis_index("s") * _NC + lax.axis_index("c")
        pltpu.sync_copy(inv_hbm, inv_v)
        inv = inv_v[...]

        def process(base, n, n_vec, clamp):
            pltpu.sync_copy(kept_hbm.at[pl.ds(base, n)], idx_v.at[pl.ds(0, n)])
            first = idx_v[pl.ds(0, _LANES)][0]
            ls = max(0, n - _LANES)
            last = idx_v[pl.ds(ls, _LANES)][n - 1 - ls]
            fa = jnp.minimum(first >> 3, (E - S) // 8) * 8
            fast = (last - fa) < S

            @pl.when(fast)
            def _fast():
                cps = (
                    pltpu.async_copy(vals_hbm.at[pl.ds(fa, S)], st_v, sg),
                    pltpu.async_copy(row0_hbm.at[pl.ds(fa, S)], st_r0, sg),
                    pltpu.async_copy(row1_hbm.at[pl.ds(fa, S)], st_r1, sg),
                )
                for cp in cps:
                    cp.wait()

                def body(s):
                    li = idx_v[pl.ds(s, _LANES)] - fa
                    if clamp:
                        li = jnp.clip(li, 0, S - 1)
                    ov_v[pl.ds(s, _LANES)] = plsc.load_gather(st_v, [li]) * inv
                    ov_r0[pl.ds(s, _LANES)] = plsc.load_gather(st_r0, [li])
                    ov_r1[pl.ds(s, _LANES)] = plsc.load_gather(st_r1, [li])
                _vec_loop(n_vec, body)

            @pl.when(jnp.logical_not(fast))
            def _slow():
                cps = (
                    pltpu.async_copy(vals_hbm.at[idx_v.at[pl.ds(0, n)]],
                                     ov_v.at[pl.ds(0, n)], sg),
                    pltpu.async_copy(row0_hbm.at[idx_v.at[pl.ds(0, n)]],
                                     ov_r0.at[pl.ds(0, n)], sg),
                    pltpu.async_copy(row1_hbm.at[idx_v.at[pl.ds(0, n)]],
                                     ov_r1.at[pl.ds(0, n)], sg),
                )
                for cp in cps:
                    cp.wait()

                def scale(s):
                    ov_v[pl.ds(s, _LANES)] = ov_v[pl.ds(s, _LANES)] * inv
                _vec_loop(n_vec, scale)

            wps = (
                pltpu.async_copy(ov_r0.at[pl.ds(0, n)],
                                 out_r0_hbm.at[pl.ds(base, n)], sw),
                pltpu.async_copy(ov_r1.at[pl.ds(0, n)],
                                 out_r1_hbm.at[pl.ds(base, n)], sw),
                pltpu.async_copy(ov_v.at[pl.ds(0, n)],
                                 out_vals_hbm.at[pl.ds(base, n)], sw),
            )
            for wp in wps:
                wp.wait()

        n_base = nc_full // _NW
        n_extra = nc_full % _NW
        n_w = n_base + jnp.where(w < n_extra, 1, 0)

        def chunk_body(i, carry):
            process((w + i * _NW) * C, C, n_vec_c, False)
            return carry
        lax.fori_loop(0, n_w, chunk_body, 0, unroll=False)

        if rem > 0:
            @pl.when(w == _NW - 1)
            def _tail():
                process(nc_full * C, rem, n_vec_t, True)

    return sc_kernel


def kernel(vals, idxs, kept_idx, keepRate):
    E = vals.shape[0]
    K = kept_idx.shape[0]
    inv = jnp.full((_LANES,), 1.0, dtype=jnp.float32) / jnp.asarray(
        keepRate, dtype=jnp.float32)
    out_r0, out_r1, out_vals = _build(E, K)(
        vals, idxs[0], idxs[1], kept_idx, inv)
    return (jnp.stack([out_r0, out_r1]), out_vals)
